# Initial kernel scaffold; baseline (speedup 1.0000x reference)
import jax
import jax.numpy as jnp
from jax.experimental import pallas as pl


def kernel(queries, keys, train_labels):
    raise NotImplementedError("write your pallas kernel here")


# speedup vs baseline: 2.2190x; 2.2190x over previous
import functools

import jax
import jax.numpy as jnp
from jax import lax
from jax.experimental import pallas as pl
from jax.experimental.pallas import tpu as pltpu
from jax.experimental.pallas import tpu_sc as plsc

Q = 1024
K = 100000
D = 16
C = 14
TOPN = 10
EPS = 1e-8

TILE = 4096
NT = 25
KPAD = TILE * NT
GSZ = 32
GPT = TILE // GSZ
NG = NT * GPT
S = 16
NCAND = S * GSZ

BQ = 256
BQ3 = 8


def _k1_body(k_ref, q_ref, gm_ref):
    kt = k_ref[...]
    q = q_ref[...]
    kn = jnp.sqrt(jnp.sum(kt * kt, axis=1, keepdims=True))
    qn = jnp.sqrt(jnp.sum(q * q, axis=0, keepdims=True))
    dot = jnp.dot(kt, q, preferred_element_type=jnp.float32)
    denom = jnp.maximum(kn * qn, jnp.float32(EPS))
    dist = 1.0 - dot / denom
    t = pl.program_id(0)
    row = t * TILE + lax.broadcasted_iota(jnp.int32, (TILE, 1), 0)
    dist = jnp.where(row < K, dist, jnp.float32(jnp.inf))
    gm = jnp.min(dist.reshape(GPT, GSZ, BQ), axis=1)
    gm_ref[...] = gm


def _k2_body(gm_ref, sel_ref):
    x = gm_ref[...]
    riota = lax.broadcasted_iota(jnp.int32, (NG, BQ), 0)
    siota = lax.broadcasted_iota(jnp.int32, (S, BQ), 0)
    sel = jnp.zeros((S, BQ), jnp.int32)
    for s in range(S):
        m = jnp.min(x, axis=0, keepdims=True)
        idx = jnp.min(jnp.where(x == m, riota, NG), axis=0, keepdims=True)
        sel = jnp.where(siota == s, idx, sel)
        x = jnp.where(riota == idx, jnp.float32(3.0e38), x)
    sel_ref[...] = sel


def _k3b_body(q_ref, cand_ref, cd_ref):
    q = q_ref[...]
    ck = cand_ref[...]
    qn = jnp.sqrt(jnp.sum(q * q, axis=1, keepdims=True))
    kn2 = jnp.sum(ck * ck, axis=1, keepdims=True)
    kn = jnp.sqrt(kn2).reshape(1, BQ3 * NCAND)
    dot = lax.dot_general(q, ck, (((1,), (1,)), ((), ())),
                          preferred_element_type=jnp.float32)
    denom = jnp.maximum(qn * kn, jnp.float32(EPS))
    dist = 1.0 - dot / denom
    lane = lax.broadcasted_iota(jnp.int32, (BQ3, BQ3 * NCAND), 1)
    rowi = lax.broadcasted_iota(jnp.int32, (BQ3, BQ3 * NCAND), 0)
    own = (lane // NCAND) == rowi
    dz = jnp.where(own, dist, jnp.float32(0.0))
    acc = dz[:, 0:NCAND]
    for b in range(1, BQ3):
        acc = acc + dz[:, b * NCAND:(b + 1) * NCAND]
    cd_ref[...] = acc


def _sc_params():
    return pltpu.CompilerParams(needs_layout_passes=False)


def _make_k3a(nc, ns):
    nw = nc * ns
    qpw = Q // nw
    mesh = plsc.VectorSubcoreMesh(core_axis_name="c", subcore_axis_name="s")

    @functools.partial(
        pl.kernel,
        out_type=jax.ShapeDtypeStruct((Q, 64, 128), jnp.float32),
        mesh=mesh,
        compiler_params=_sc_params(),
        scratch_types=[
            pltpu.VMEM((qpw, 16), jnp.int32),
            pltpu.VMEM((64,), jnp.int32),
            pltpu.VMEM((64, 128), jnp.float32),
            pltpu.SemaphoreType.DMA,
        ],
    )
    def k3a(keys_hbm, sel_hbm, cand_hbm, sel32, cidx, cand, sem):
        w = lax.axis_index("s") * nc + lax.axis_index("c")
        base_q = w * qpw
        pltpu.sync_copy(sel_hbm.at[pl.ds(base_q, qpw)], sel32)
        iota = lax.iota(jnp.int32, 16)

        def full_i(v):
            return jnp.full((16,), v, jnp.int32)

        def per_query(qi, _):
            selv = plsc.load_gather(sel32, [full_i(qi), iota])
            for r in range(4):
                plsc.store_scatter(cidx, [iota * 4 + r], selv * 4 + r)
            pltpu.async_copy(keys_hbm.at[cidx], cand, sem).wait()
            pltpu.sync_copy(cand, cand_hbm.at[base_q + qi])
            return 0

        lax.fori_loop(0, qpw, per_query, 0)

    return k3a


def _make_k3c(nc, ns):
    nw = nc * ns
    qpw = Q // nw
    mesh = plsc.VectorSubcoreMesh(core_axis_name="c", subcore_axis_name="s")

    @functools.partial(
        pl.kernel,
        out_type=(
            jax.ShapeDtypeStruct((nw, qpw, 16), jnp.float32),
            jax.ShapeDtypeStruct((nw, qpw, 16), jnp.float32),
            jax.ShapeDtypeStruct((nw, qpw, 16), jnp.float32),
        ),
        mesh=mesh,
        compiler_params=_sc_params(),
        scratch_types=[
            pltpu.VMEM((qpw, 16), jnp.int32),
            pltpu.VMEM((4, 128), jnp.float32),
            pltpu.VMEM((16,), jnp.int32),
            pltpu.VMEM((16, 128), jnp.int32),
            pltpu.VMEM((qpw, 16), jnp.float32),
            pltpu.VMEM((qpw, 16), jnp.float32),
            pltpu.VMEM((qpw, 16), jnp.float32),
            pltpu.SemaphoreType.DMA,
        ],
    )
    def k3c(cd_hbm, sel_hbm, lab_hbm,
            topd_hbm, votes_hbm, pos_hbm,
            sel32, cdb, winrow, lab, od_, ov_, op_, sem):
        w = lax.axis_index("s") * nc + lax.axis_index("c")
        base_q = w * qpw
        pltpu.sync_copy(sel_hbm.at[pl.ds(base_q, qpw)], sel32)
        iota = lax.iota(jnp.int32, 16)
        inf = jnp.float32(jnp.inf)
        bigi = jnp.int32(2 ** 30)

        def full_i(v):
            return jnp.full((16,), v, jnp.int32)

        def per_query(qi, _):
            pltpu.sync_copy(cd_hbm.at[base_q + qi], cdb)
            selv = plsc.load_gather(sel32, [full_i(qi), iota])
            basev = selv * GSZ

            def rows_lanes(o):
                c = iota * GSZ + o
                return c >> 7, c & 127

            def top_body(t, carry):
                od_row, win_row = carry

                def p1(o, m):
                    r, l = rows_lanes(o)
                    return jnp.minimum(m, plsc.load_gather(cdb, [r, l]))
                m16 = lax.fori_loop(0, GSZ, p1, jnp.full((16,), inf))
                m = jnp.min(m16)

                def p2(o, g):
                    r, l = rows_lanes(o)
                    dj = plsc.load_gather(cdb, [r, l])
                    return jnp.minimum(g, jnp.where(dj == m, basev + o, bigi))
                g16 = lax.fori_loop(0, GSZ, p2, jnp.full((16,), bigi))
                g = jnp.min(g16)

                def p3(o, _m):
                    r, l = rows_lanes(o)
                    dj = plsc.load_gather(cdb, [r, l])
                    hit = (dj == m) & ((basev + o) == g)
                    plsc.store_scatter(cdb, [r, l], jnp.where(hit, inf, dj))
                    return 0
                lax.fori_loop(0, GSZ, p3, 0)

                lt = iota == t
                od_row = jnp.where(lt, m, od_row)
                win_row = jnp.where(lt, g, win_row)
                return od_row, win_row
            od_row, win_row = lax.fori_loop(
                0, TOPN, top_body,
                (jnp.full((16,), inf), jnp.zeros((16,), jnp.int32)))

            winrow[...] = win_row >> 3
            pltpu.async_copy(lab_hbm.at[winrow], lab, sem).wait()
            colb16 = (win_row & 7) * 16

            def lab_body(i, acc):
                cb = jnp.min(jnp.where(iota == i, colb16, bigi))
                row = plsc.load_gather(lab, [full_i(i), cb + iota])
                return acc + row.astype(jnp.float32)
            vacc = lax.fori_loop(0, TOPN, lab_body, jnp.zeros((16,), jnp.float32))
            votes = vacc / jnp.float32(TOPN)
            posv = jnp.where(votes > 0.5, jnp.float32(1.0), jnp.float32(0.0))

            plsc.store_scatter(od_, [full_i(qi), iota], od_row)
            plsc.store_scatter(ov_, [full_i(qi), iota], votes)
            plsc.store_scatter(op_, [full_i(qi), iota], posv)
            return 0

        lax.fori_loop(0, qpw, per_query, 0)
        pltpu.sync_copy(od_, topd_hbm.at[w])
        pltpu.sync_copy(ov_, votes_hbm.at[w])
        pltpu.sync_copy(op_, pos_hbm.at[w])

    return k3c


def kernel(queries, keys, train_labels):
    keys_pad = jnp.pad(keys, ((0, KPAD - K), (0, 0)))
    keys128 = keys_pad.reshape(KPAD * D // 128, 128)
    queries_t = queries.T
    lab_pad = jnp.pad(train_labels, ((0, 0), (0, 16 - C)))
    lab128 = lab_pad.reshape(K * 16 // 128, 128)

    gm = pl.pallas_call(
        _k1_body,
        grid=(NT, Q // BQ),
        in_specs=[
            pl.BlockSpec((TILE, D), lambda t, qb: (t, 0)),
            pl.BlockSpec((D, BQ), lambda t, qb: (0, qb)),
        ],
        out_specs=pl.BlockSpec((GPT, BQ), lambda t, qb: (t, qb)),
        out_shape=jax.ShapeDtypeStruct((NG, Q), jnp.float32),
    )(keys_pad, queries_t)

    sel_t = pl.pallas_call(
        _k2_body,
        grid=(Q // BQ,),
        in_specs=[pl.BlockSpec((NG, BQ), lambda qb: (0, qb))],
        out_specs=pl.BlockSpec((S, BQ), lambda qb: (0, qb)),
        out_shape=jax.ShapeDtypeStruct((S, Q), jnp.int32),
    )(gm)
    sel = sel_t.T

    try:
        info = plsc.get_sparse_core_info()
        nc, ns = info.num_cores, info.num_subcores
    except Exception:
        nc, ns = 2, 16

    cand = _make_k3a(nc, ns)(keys128, sel)
    cand_rows = cand.reshape(Q * NCAND, D)

    cdist = pl.pallas_call(
        _k3b_body,
        grid=(Q // BQ3,),
        in_specs=[
            pl.BlockSpec((BQ3, D), lambda b: (b, 0)),
            pl.BlockSpec((BQ3 * NCAND, D), lambda b: (b, 0)),
        ],
        out_specs=pl.BlockSpec((BQ3, NCAND), lambda b: (b, 0)),
        out_shape=jax.ShapeDtypeStruct((Q, NCAND), jnp.float32),
    )(queries, cand_rows)
    cd128 = cdist.reshape(Q, 4, 128)

    topd, votes, pos = _make_k3c(nc, ns)(cd128, sel, lab128)

    topd = topd.reshape(Q, 16)[:, :TOPN]
    votes = votes.reshape(Q, 16)[:, :C]
    pos = pos.reshape(Q, 16)[:, :C]
    return topd, votes, pos


# speedup vs baseline: 2.3048x; 1.0386x over previous
import functools

import jax
import jax.numpy as jnp
from jax import lax
from jax.experimental import pallas as pl
from jax.experimental.pallas import tpu as pltpu
from jax.experimental.pallas import tpu_sc as plsc

Q = 1024
K = 100000
D = 16
C = 14
TOPN = 10
EPS = 1e-8

TILE = 4096
NT = 25
KPAD = TILE * NT
GSZ = 32
GPT = TILE // GSZ
NG = NT * GPT
S = 16
NCAND = S * GSZ

BQ = 256
BQ3 = 8


def _k1_body(k_ref, q_ref, gm_ref):
    kt = k_ref[...]
    q = q_ref[...]
    kn = jnp.sqrt(jnp.sum(kt * kt, axis=1, keepdims=True))
    rinv = 1.0 / jnp.maximum(kn, jnp.float32(1e-30))
    dot = jnp.dot(kt, q, preferred_element_type=jnp.float32)
    s = dot * rinv
    t = pl.program_id(0)
    row = t * TILE + lax.broadcasted_iota(jnp.int32, (TILE, 1), 0)
    s = jnp.where(row < K, s, jnp.float32(-3.0e38))
    gm_ref[...] = jnp.max(s.reshape(GPT, GSZ, BQ), axis=1)


def _k2_body(gm_ref, sel_ref):
    x = gm_ref[...]
    riota = lax.broadcasted_iota(jnp.int32, (NG, BQ), 0)
    siota = lax.broadcasted_iota(jnp.int32, (S, BQ), 0)
    sel = jnp.zeros((S, BQ), jnp.int32)
    for s in range(S):
        m = jnp.max(x, axis=0, keepdims=True)
        idx = jnp.min(jnp.where(x == m, riota, NG), axis=0, keepdims=True)
        sel = jnp.where(siota == s, idx, sel)
        x = jnp.where(riota == idx, jnp.float32(-3.0e38), x)
    sel_ref[...] = sel


def _k3b_body(q_ref, cand_ref, sel_ref, topd_ref, win_ref):
    q = q_ref[...]
    ck = cand_ref[...]
    qn = jnp.sqrt(jnp.sum(q * q, axis=1, keepdims=True))
    kn = jnp.sqrt(jnp.sum(ck * ck, axis=1, keepdims=True)).reshape(1, BQ3 * NCAND)
    dot = lax.dot_general(q, ck, (((1,), (1,)), ((), ())),
                          preferred_element_type=jnp.float32)
    denom = jnp.maximum(qn * kn, jnp.float32(EPS))
    dist = 1.0 - dot / denom
    lane = lax.broadcasted_iota(jnp.int32, (BQ3, BQ3 * NCAND), 1)
    rowi = lax.broadcasted_iota(jnp.int32, (BQ3, BQ3 * NCAND), 0)
    dz = jnp.where((lane // NCAND) == rowi, dist, jnp.float32(0.0))
    x = dz[:, 0:NCAND]
    for b in range(1, BQ3):
        x = x + dz[:, b * NCAND:(b + 1) * NCAND]

    selb = sel_ref[...]
    soff = lax.broadcasted_iota(jnp.int32, (BQ3, NCAND), 1)
    grp = soff // GSZ
    gidx = jnp.zeros((BQ3, NCAND), jnp.int32)
    for s in range(S):
        gidx = gidx + selb[:, s:s + 1] * (grp == s).astype(jnp.int32)
    gidx = gidx * GSZ + (soff % GSZ)

    inf = jnp.float32(jnp.inf)
    bigi = jnp.int32(2 ** 30)
    lane16 = lax.broadcasted_iota(jnp.int32, (BQ3, 16), 1)
    topd = jnp.zeros((BQ3, 16), jnp.float32)
    win = jnp.zeros((BQ3, 16), jnp.int32)
    for t in range(TOPN):
        m = jnp.min(x, axis=1, keepdims=True)
        gs = jnp.min(jnp.where(x == m, gidx, bigi), axis=1, keepdims=True)
        topd = jnp.where(lane16 == t, m, topd)
        win = jnp.where(lane16 == t, gs, win)
        x = jnp.where((x == m) & (gidx == gs), inf, x)
    topd_ref[...] = topd
    win_ref[...] = win


def _sc_params():
    return pltpu.CompilerParams(needs_layout_passes=False)


def _make_k3a(nc, ns):
    nw = nc * ns
    qpw = Q // nw
    mesh = plsc.VectorSubcoreMesh(core_axis_name="c", subcore_axis_name="s")

    @functools.partial(
        pl.kernel,
        out_type=jax.ShapeDtypeStruct((Q, 64, 128), jnp.float32),
        mesh=mesh,
        compiler_params=_sc_params(),
        scratch_types=[
            pltpu.VMEM((qpw, 16), jnp.int32),
            pltpu.VMEM((64,), jnp.int32),
            pltpu.VMEM((64, 128), jnp.float32),
            pltpu.SemaphoreType.DMA,
        ],
    )
    def k3a(keys_hbm, sel_hbm, cand_hbm, sel32, cidx, cand, sem):
        w = lax.axis_index("s") * nc + lax.axis_index("c")
        base_q = w * qpw
        pltpu.sync_copy(sel_hbm.at[pl.ds(base_q, qpw)], sel32)
        iota = lax.iota(jnp.int32, 16)

        def full_i(v):
            return jnp.full((16,), v, jnp.int32)

        def per_query(qi, _):
            selv = plsc.load_gather(sel32, [full_i(qi), iota])
            for r in range(4):
                plsc.store_scatter(cidx, [iota * 4 + r], selv * 4 + r)
            pltpu.async_copy(keys_hbm.at[cidx], cand, sem).wait()
            pltpu.sync_copy(cand, cand_hbm.at[base_q + qi])
            return 0

        lax.fori_loop(0, qpw, per_query, 0)

    return k3a


def _make_k3e(nc, ns):
    nw = nc * ns
    qpw = Q // nw
    mesh = plsc.VectorSubcoreMesh(core_axis_name="c", subcore_axis_name="s")

    @functools.partial(
        pl.kernel,
        out_type=(
            jax.ShapeDtypeStruct((nw, qpw, 16), jnp.float32),
            jax.ShapeDtypeStruct((nw, qpw, 16), jnp.float32),
        ),
        mesh=mesh,
        compiler_params=_sc_params(),
        scratch_types=[
            pltpu.VMEM((qpw, 16), jnp.int32),
            pltpu.VMEM((4, 128), jnp.int32),
            pltpu.VMEM((qpw * 16, 128), jnp.int32),
            pltpu.VMEM((qpw, 16), jnp.float32),
            pltpu.VMEM((qpw, 16), jnp.float32),
            pltpu.SemaphoreType.DMA,
        ],
    )
    def k3e(win_hbm, lab_hbm, votes_hbm, pos_hbm,
            win32, rows_idx, labbuf, ov_, op_, sem):
        w = lax.axis_index("s") * nc + lax.axis_index("c")
        base_q = w * qpw
        pltpu.sync_copy(win_hbm.at[pl.ds(base_q, qpw)], win32)
        iota = lax.iota(jnp.int32, 16)

        def full_i(v):
            return jnp.full((16,), v, jnp.int32)

        def stage(qi, _):
            wrow = plsc.load_gather(win32, [full_i(qi), iota]) >> 3
            p = qi * 16 + iota
            plsc.store_scatter(rows_idx, [p >> 7, p & 127], wrow)
            return 0
        lax.fori_loop(0, qpw, stage, 0)

        copies = [
            pltpu.async_copy(lab_hbm.at[rows_idx.at[cc]],
                             labbuf.at[pl.ds(cc * 128, 128)], sem)
            for cc in range(4)
        ]
        for cp in copies:
            cp.wait()

        def per_query(qi, _):
            def lab_body(i, acc):
                wv = plsc.load_gather(win32, [full_i(qi), full_i(i)])
                cb = (wv & 7) * 16
                row = plsc.load_gather(labbuf, [full_i(qi * 16 + i), cb + iota])
                return acc + row.astype(jnp.float32)
            vacc = lax.fori_loop(0, TOPN, lab_body, jnp.zeros((16,), jnp.float32))
            votes = vacc / jnp.float32(TOPN)
            posv = jnp.where(votes > 0.5, jnp.float32(1.0), jnp.float32(0.0))
            plsc.store_scatter(ov_, [full_i(qi), iota], votes)
            plsc.store_scatter(op_, [full_i(qi), iota], posv)
            return 0
        lax.fori_loop(0, qpw, per_query, 0)

        pltpu.sync_copy(ov_, votes_hbm.at[w])
        pltpu.sync_copy(op_, pos_hbm.at[w])

    return k3e


def kernel(queries, keys, train_labels):
    keys_pad = jnp.pad(keys, ((0, KPAD - K), (0, 0)))
    keys128 = keys_pad.reshape(KPAD * D // 128, 128)
    queries_t = queries.T
    lab_pad = jnp.pad(train_labels, ((0, 0), (0, 16 - C)))
    lab128 = lab_pad.reshape(K * 16 // 128, 128)

    gm = pl.pallas_call(
        _k1_body,
        grid=(NT, Q // BQ),
        in_specs=[
            pl.BlockSpec((TILE, D), lambda t, qb: (t, 0)),
            pl.BlockSpec((D, BQ), lambda t, qb: (0, qb)),
        ],
        out_specs=pl.BlockSpec((GPT, BQ), lambda t, qb: (t, qb)),
        out_shape=jax.ShapeDtypeStruct((NG, Q), jnp.float32),
    )(keys_pad, queries_t)

    sel_t = pl.pallas_call(
        _k2_body,
        grid=(Q // BQ,),
        in_specs=[pl.BlockSpec((NG, BQ), lambda qb: (0, qb))],
        out_specs=pl.BlockSpec((S, BQ), lambda qb: (0, qb)),
        out_shape=jax.ShapeDtypeStruct((S, Q), jnp.int32),
    )(gm)
    sel = sel_t.T

    try:
        info = plsc.get_sparse_core_info()
        nc, ns = info.num_cores, info.num_subcores
    except Exception:
        nc, ns = 2, 16

    cand = _make_k3a(nc, ns)(keys128, sel)
    cand_rows = cand.reshape(Q * NCAND, D)

    topd16, win = pl.pallas_call(
        _k3b_body,
        grid=(Q // BQ3,),
        in_specs=[
            pl.BlockSpec((BQ3, D), lambda b: (b, 0)),
            pl.BlockSpec((BQ3 * NCAND, D), lambda b: (b, 0)),
            pl.BlockSpec((BQ3, S), lambda b: (b, 0)),
        ],
        out_specs=(
            pl.BlockSpec((BQ3, 16), lambda b: (b, 0)),
            pl.BlockSpec((BQ3, 16), lambda b: (b, 0)),
        ),
        out_shape=(
            jax.ShapeDtypeStruct((Q, 16), jnp.float32),
            jax.ShapeDtypeStruct((Q, 16), jnp.int32),
        ),
    )(queries, cand_rows, sel)

    votes, pos = _make_k3e(nc, ns)(win, lab128)

    topd = topd16[:, :TOPN]
    votes = votes.reshape(Q, 16)[:, :C]
    pos = pos.reshape(Q, 16)[:, :C]
    return topd, votes, pos


# speedup vs baseline: 2.3165x; 1.0051x over previous
import functools

import jax
import jax.numpy as jnp
from jax import lax
from jax.experimental import pallas as pl
from jax.experimental.pallas import tpu as pltpu
from jax.experimental.pallas import tpu_sc as plsc

Q = 1024
K = 100000
D = 16
C = 14
TOPN = 10
EPS = 1e-8

TILE = 4096
NT = 25
KPAD = TILE * NT
GSZ = 32
GPT = TILE // GSZ
NG = NT * GPT
S = 16
NCAND = S * GSZ

BQ = 256
BQ3 = 8


def _k1_body(k_ref, q_ref, gm_ref):
    kt = k_ref[...]
    q = q_ref[...]
    kn = jnp.sqrt(jnp.sum(kt * kt, axis=1, keepdims=True))
    rinv = 1.0 / jnp.maximum(kn, jnp.float32(1e-30))
    dot = jnp.dot(kt, q, preferred_element_type=jnp.float32)
    s = dot * rinv
    t = pl.program_id(0)
    row = t * TILE + lax.broadcasted_iota(jnp.int32, (TILE, 1), 0)
    s = jnp.where(row < K, s, jnp.float32(-3.0e38))
    gm_ref[...] = jnp.max(s.reshape(GPT, GSZ, BQ), axis=1)


def _k2_body(gm_ref, sel_ref):
    x = gm_ref[...]
    riota = lax.broadcasted_iota(jnp.int32, (NG, BQ), 0)
    siota = lax.broadcasted_iota(jnp.int32, (S, BQ), 0)
    sel = jnp.zeros((S, BQ), jnp.int32)
    for s in range(S):
        m = jnp.max(x, axis=0, keepdims=True)
        idx = jnp.min(jnp.where(x == m, riota, NG), axis=0, keepdims=True)
        sel = jnp.where(siota == s, idx, sel)
        x = jnp.where(riota == idx, jnp.float32(-3.0e38), x)
    sel_ref[...] = sel


def _k3b_body(q_ref, cand_ref, clab_ref, sel_ref, topd_ref, votes_ref, pos_ref):
    q = q_ref[...]
    ck = cand_ref[...]
    qn = jnp.sqrt(jnp.sum(q * q, axis=1, keepdims=True))
    kn = jnp.sqrt(jnp.sum(ck * ck, axis=1, keepdims=True)).reshape(1, BQ3 * NCAND)
    dot = lax.dot_general(q, ck, (((1,), (1,)), ((), ())),
                          preferred_element_type=jnp.float32)
    denom = jnp.maximum(qn * kn, jnp.float32(EPS))
    dist = 1.0 - dot / denom
    lane = lax.broadcasted_iota(jnp.int32, (BQ3, BQ3 * NCAND), 1)
    rowi = lax.broadcasted_iota(jnp.int32, (BQ3, BQ3 * NCAND), 0)
    own = (lane // NCAND) == rowi
    dz = jnp.where(own, dist, jnp.float32(0.0))
    x = dz[:, 0:NCAND]
    for b in range(1, BQ3):
        x = x + dz[:, b * NCAND:(b + 1) * NCAND]

    selb = sel_ref[...]
    soff = lax.broadcasted_iota(jnp.int32, (BQ3, NCAND), 1)
    grp = soff // GSZ
    gidx = jnp.zeros((BQ3, NCAND), jnp.int32)
    for s in range(S):
        gidx = gidx + selb[:, s:s + 1] * (grp == s).astype(jnp.int32)
    gidx = gidx * GSZ + (soff % GSZ)

    inf = jnp.float32(jnp.inf)
    bigi = jnp.int32(2 ** 30)
    lane16 = lax.broadcasted_iota(jnp.int32, (BQ3, 16), 1)
    topd = jnp.zeros((BQ3, 16), jnp.float32)
    hsum = jnp.zeros((BQ3, NCAND), jnp.float32)
    for t in range(TOPN):
        m = jnp.min(x, axis=1, keepdims=True)
        gs = jnp.min(jnp.where(x == m, gidx, bigi), axis=1, keepdims=True)
        topd = jnp.where(lane16 == t, m, topd)
        hit = (x == m) & (gidx == gs)
        hsum = hsum + jnp.where(hit, jnp.float32(1.0), jnp.float32(0.0))
        x = jnp.where(hit, inf, x)
    topd_ref[...] = topd

    hfull = jnp.concatenate([hsum] * BQ3, axis=1)
    hown = jnp.where(own, hfull, jnp.float32(0.0))
    labf = clab_ref[...].astype(jnp.float32)
    vsum = jnp.dot(hown, labf, preferred_element_type=jnp.float32)
    votes = vsum / jnp.float32(TOPN)
    votes_ref[...] = votes
    pos_ref[...] = jnp.where(votes > 0.5, jnp.float32(1.0), jnp.float32(0.0))


def _sc_params():
    return pltpu.CompilerParams(needs_layout_passes=False)


def _make_k3a(nc, ns):
    nw = nc * ns
    qpw = Q // nw
    mesh = plsc.VectorSubcoreMesh(core_axis_name="c", subcore_axis_name="s")

    @functools.partial(
        pl.kernel,
        out_type=(
            jax.ShapeDtypeStruct((Q, 64, 128), jnp.float32),
            jax.ShapeDtypeStruct((Q, 64, 128), jnp.int32),
        ),
        mesh=mesh,
        compiler_params=_sc_params(),
        scratch_types=[
            pltpu.VMEM((qpw, 16), jnp.int32),
            pltpu.VMEM((64,), jnp.int32),
            pltpu.VMEM((64, 128), jnp.float32),
            pltpu.VMEM((64, 128), jnp.int32),
            pltpu.SemaphoreType.DMA,
            pltpu.SemaphoreType.DMA,
        ],
    )
    def k3a(keys_hbm, lab_hbm, sel_hbm, cand_hbm, clab_hbm,
            sel32, cidx, cand, clab, sem, sem2):
        w = lax.axis_index("s") * nc + lax.axis_index("c")
        base_q = w * qpw
        pltpu.sync_copy(sel_hbm.at[pl.ds(base_q, qpw)], sel32)
        iota = lax.iota(jnp.int32, 16)

        def full_i(v):
            return jnp.full((16,), v, jnp.int32)

        def per_query(qi, _):
            selv = plsc.load_gather(sel32, [full_i(qi), iota])
            for r in range(4):
                plsc.store_scatter(cidx, [iota * 4 + r], selv * 4 + r)
            cp1 = pltpu.async_copy(keys_hbm.at[cidx], cand, sem)
            cp2 = pltpu.async_copy(lab_hbm.at[cidx], clab, sem2)
            cp1.wait()
            cp2.wait()
            pltpu.sync_copy(cand, cand_hbm.at[base_q + qi])
            pltpu.sync_copy(clab, clab_hbm.at[base_q + qi])
            return 0

        lax.fori_loop(0, qpw, per_query, 0)

    return k3a


def kernel(queries, keys, train_labels):
    keys_pad = jnp.pad(keys, ((0, KPAD - K), (0, 0)))
    keys128 = keys_pad.reshape(KPAD * D // 128, 128)
    queries_t = queries.T
    lab_pad = jnp.pad(train_labels, ((0, 0), (0, 16 - C)))
    lab128 = lab_pad.reshape(K * 16 // 128, 128)

    gm = pl.pallas_call(
        _k1_body,
        grid=(NT, Q // BQ),
        in_specs=[
            pl.BlockSpec((TILE, D), lambda t, qb: (t, 0)),
            pl.BlockSpec((D, BQ), lambda t, qb: (0, qb)),
        ],
        out_specs=pl.BlockSpec((GPT, BQ), lambda t, qb: (t, qb)),
        out_shape=jax.ShapeDtypeStruct((NG, Q), jnp.float32),
    )(keys_pad, queries_t)

    sel_t = pl.pallas_call(
        _k2_body,
        grid=(Q // BQ,),
        in_specs=[pl.BlockSpec((NG, BQ), lambda qb: (0, qb))],
        out_specs=pl.BlockSpec((S, BQ), lambda qb: (0, qb)),
        out_shape=jax.ShapeDtypeStruct((S, Q), jnp.int32),
    )(gm)
    sel = sel_t.T

    try:
        info = plsc.get_sparse_core_info()
        nc, ns = info.num_cores, info.num_subcores
    except Exception:
        nc, ns = 2, 16

    cand, clab = _make_k3a(nc, ns)(keys128, lab128, sel)
    cand_rows = cand.reshape(Q * NCAND, D)
    clab_rows = clab.reshape(Q * NCAND, 16)

    topd16, votes16, pos16 = pl.pallas_call(
        _k3b_body,
        grid=(Q // BQ3,),
        in_specs=[
            pl.BlockSpec((BQ3, D), lambda b: (b, 0)),
            pl.BlockSpec((BQ3 * NCAND, D), lambda b: (b, 0)),
            pl.BlockSpec((BQ3 * NCAND, 16), lambda b: (b, 0)),
            pl.BlockSpec((BQ3, S), lambda b: (b, 0)),
        ],
        out_specs=(
            pl.BlockSpec((BQ3, 16), lambda b: (b, 0)),
            pl.BlockSpec((BQ3, 16), lambda b: (b, 0)),
            pl.BlockSpec((BQ3, 16), lambda b: (b, 0)),
        ),
        out_shape=(
            jax.ShapeDtypeStruct((Q, 16), jnp.float32),
            jax.ShapeDtypeStruct((Q, 16), jnp.float32),
            jax.ShapeDtypeStruct((Q, 16), jnp.float32),
        ),
    )(queries, cand_rows, clab_rows, sel)

    topd = topd16[:, :TOPN]
    votes = votes16[:, :C]
    pos = pos16[:, :C]
    return topd, votes, pos


# speedup vs baseline: 2.4800x; 1.0706x over previous
import functools

import jax
import jax.numpy as jnp
from jax import lax
from jax.experimental import pallas as pl
from jax.experimental.pallas import tpu as pltpu
from jax.experimental.pallas import tpu_sc as plsc

Q = 1024
K = 100000
D = 16
C = 14
TOPN = 10
EPS = 1e-8

TILE = 4096
NT = 25
KPAD = TILE * NT
GSZ = 32
GPT = TILE // GSZ
NG = NT * GPT
S = 16
NCAND = S * GSZ

BQ = 256
BQ3 = 8


def _k1_body(k_ref, q_ref, gm_ref):
    kt = k_ref[...]
    q = q_ref[...]
    kn = jnp.sqrt(jnp.sum(kt * kt, axis=1, keepdims=True))
    rinv = 1.0 / jnp.maximum(kn, jnp.float32(1e-30))
    dot = jnp.dot(kt, q, preferred_element_type=jnp.float32)
    s = dot * rinv
    t = pl.program_id(0)
    row = t * TILE + lax.broadcasted_iota(jnp.int32, (TILE, 1), 0)
    s = jnp.where(row < K, s, jnp.float32(-3.0e38))
    gm_ref[...] = jnp.max(s.reshape(GPT, GSZ, BQ), axis=1)


def _k2_body(gm_ref, sel_ref):
    x = gm_ref[...]
    riota = lax.broadcasted_iota(jnp.int32, (NG, BQ), 0)
    siota = lax.broadcasted_iota(jnp.int32, (S, BQ), 0)
    sel = jnp.zeros((S, BQ), jnp.int32)
    for s in range(S):
        m = jnp.max(x, axis=0, keepdims=True)
        idx = jnp.min(jnp.where(x == m, riota, NG), axis=0, keepdims=True)
        sel = jnp.where(siota == s, idx, sel)
        x = jnp.where(riota == idx, jnp.float32(-3.0e38), x)
    sel_ref[...] = sel


def _k3b_body(q_ref, cand_ref, clab_ref, sel_ref, topd_ref, votes_ref, pos_ref):
    q = q_ref[...]
    ckt = cand_ref[...]
    qn = jnp.sqrt(jnp.sum(q * q, axis=1, keepdims=True))
    kn = jnp.sqrt(jnp.sum(ckt * ckt, axis=0, keepdims=True))
    dot = jnp.dot(q, ckt, preferred_element_type=jnp.float32)
    denom = jnp.maximum(qn * kn, jnp.float32(EPS))
    dist = 1.0 - dot / denom
    lane = lax.broadcasted_iota(jnp.int32, (BQ3, BQ3 * NCAND), 1)
    rowi = lax.broadcasted_iota(jnp.int32, (BQ3, BQ3 * NCAND), 0)
    own = (lane // NCAND) == rowi
    dz = jnp.where(own, dist, jnp.float32(0.0))
    x = dz[:, 0:NCAND]
    for b in range(1, BQ3):
        x = x + dz[:, b * NCAND:(b + 1) * NCAND]

    selb = sel_ref[...]
    soff = lax.broadcasted_iota(jnp.int32, (BQ3, NCAND), 1)
    grp = soff // GSZ
    gidx = jnp.zeros((BQ3, NCAND), jnp.int32)
    for s in range(S):
        gidx = gidx + selb[:, s:s + 1] * (grp == s).astype(jnp.int32)
    gidx = gidx * GSZ + (soff % GSZ)

    inf = jnp.float32(jnp.inf)
    bigi = jnp.int32(2 ** 30)
    lane16 = lax.broadcasted_iota(jnp.int32, (BQ3, 16), 1)
    topd = jnp.zeros((BQ3, 16), jnp.float32)
    hsum = jnp.zeros((BQ3, NCAND), jnp.float32)
    for t in range(TOPN):
        m = jnp.min(x, axis=1, keepdims=True)
        gs = jnp.min(jnp.where(x == m, gidx, bigi), axis=1, keepdims=True)
        topd = jnp.where(lane16 == t, m, topd)
        hit = (x == m) & (gidx == gs)
        hsum = hsum + jnp.where(hit, jnp.float32(1.0), jnp.float32(0.0))
        x = jnp.where(hit, inf, x)
    topd_ref[...] = topd

    hfull = jnp.concatenate([hsum] * BQ3, axis=1)
    hown = jnp.where(own, hfull, jnp.float32(0.0))
    labf = clab_ref[...].astype(jnp.float32)
    vsum = jnp.dot(hown, labf, preferred_element_type=jnp.float32)
    votes = vsum / jnp.float32(TOPN)
    votes_ref[...] = votes
    pos_ref[...] = jnp.where(votes > 0.5, jnp.float32(1.0), jnp.float32(0.0))


def _sc_params():
    return pltpu.CompilerParams(needs_layout_passes=False)


def _make_k3a(nc, ns):
    nw = nc * ns
    qpw = Q // nw
    mesh = plsc.VectorSubcoreMesh(core_axis_name="c", subcore_axis_name="s")

    @functools.partial(
        pl.kernel,
        out_type=(
            jax.ShapeDtypeStruct((Q, 64, 128), jnp.float32),
            jax.ShapeDtypeStruct((Q, 64, 128), jnp.int32),
        ),
        mesh=mesh,
        compiler_params=_sc_params(),
        scratch_types=[
            pltpu.VMEM((qpw, 16), jnp.int32),
            pltpu.VMEM((64,), jnp.int32),
            pltpu.VMEM((64, 128), jnp.float32),
            pltpu.VMEM((64, 128), jnp.int32),
            pltpu.SemaphoreType.DMA,
            pltpu.SemaphoreType.DMA,
        ],
    )
    def k3a(keys_hbm, lab_hbm, sel_hbm, cand_hbm, clab_hbm,
            sel32, cidx, cand, clab, sem, sem2):
        w = lax.axis_index("s") * nc + lax.axis_index("c")
        base_q = w * qpw
        pltpu.sync_copy(sel_hbm.at[pl.ds(base_q, qpw)], sel32)
        iota = lax.iota(jnp.int32, 16)

        def full_i(v):
            return jnp.full((16,), v, jnp.int32)

        def per_query(qi, _):
            selv = plsc.load_gather(sel32, [full_i(qi), iota])
            for r in range(4):
                plsc.store_scatter(cidx, [iota * 4 + r], selv * 4 + r)
            cp1 = pltpu.async_copy(keys_hbm.at[cidx], cand, sem)
            cp2 = pltpu.async_copy(lab_hbm.at[cidx], clab, sem2)
            cp1.wait()
            cp2.wait()
            pltpu.sync_copy(cand, cand_hbm.at[base_q + qi])
            pltpu.sync_copy(clab, clab_hbm.at[base_q + qi])
            return 0

        lax.fori_loop(0, qpw, per_query, 0)

    return k3a


def kernel(queries, keys, train_labels):
    keys_pad = jnp.pad(keys, ((0, KPAD - K), (0, 0)))
    keys128 = keys_pad.reshape(KPAD * D // 128, 128)
    queries_t = queries.T
    lab_pad = jnp.pad(train_labels, ((0, 0), (0, 16 - C)))
    lab128 = lab_pad.reshape(K * 16 // 128, 128)

    gm = pl.pallas_call(
        _k1_body,
        grid=(NT, Q // BQ),
        in_specs=[
            pl.BlockSpec((TILE, D), lambda t, qb: (t, 0)),
            pl.BlockSpec((D, BQ), lambda t, qb: (0, qb)),
        ],
        out_specs=pl.BlockSpec((GPT, BQ), lambda t, qb: (t, qb)),
        out_shape=jax.ShapeDtypeStruct((NG, Q), jnp.float32),
    )(keys_pad, queries_t)

    sel_t = pl.pallas_call(
        _k2_body,
        grid=(Q // BQ,),
        in_specs=[pl.BlockSpec((NG, BQ), lambda qb: (0, qb))],
        out_specs=pl.BlockSpec((S, BQ), lambda qb: (0, qb)),
        out_shape=jax.ShapeDtypeStruct((S, Q), jnp.int32),
    )(gm)
    sel = sel_t.T

    try:
        info = plsc.get_sparse_core_info()
        nc, ns = info.num_cores, info.num_subcores
    except Exception:
        nc, ns = 2, 16

    cand, clab = _make_k3a(nc, ns)(keys128, lab128, sel)
    cand_t = cand.reshape(Q * NCAND, D).T
    clab_rows = clab.reshape(Q * NCAND, 16)

    topd16, votes16, pos16 = pl.pallas_call(
        _k3b_body,
        grid=(Q // BQ3,),
        in_specs=[
            pl.BlockSpec((BQ3, D), lambda b: (b, 0)),
            pl.BlockSpec((D, BQ3 * NCAND), lambda b: (0, b)),
            pl.BlockSpec((BQ3 * NCAND, 16), lambda b: (b, 0)),
            pl.BlockSpec((BQ3, S), lambda b: (b, 0)),
        ],
        out_specs=(
            pl.BlockSpec((BQ3, 16), lambda b: (b, 0)),
            pl.BlockSpec((BQ3, 16), lambda b: (b, 0)),
            pl.BlockSpec((BQ3, 16), lambda b: (b, 0)),
        ),
        out_shape=(
            jax.ShapeDtypeStruct((Q, 16), jnp.float32),
            jax.ShapeDtypeStruct((Q, 16), jnp.float32),
            jax.ShapeDtypeStruct((Q, 16), jnp.float32),
        ),
    )(queries, cand_t, clab_rows, sel)

    topd = topd16[:, :TOPN]
    votes = votes16[:, :C]
    pos = pos16[:, :C]
    return topd, votes, pos


# speedup vs baseline: 2.8698x; 1.1572x over previous
import functools

import jax
import jax.numpy as jnp
from jax import lax
from jax.experimental import pallas as pl
from jax.experimental.pallas import tpu as pltpu
from jax.experimental.pallas import tpu_sc as plsc

Q = 1024
K = 100000
D = 16
C = 14
TOPN = 10
EPS = 1e-8

TILE = 4096
NT = 25
KPAD = TILE * NT
GSZ = 32
GPT = TILE // GSZ
NG = NT * GPT
S = 16
NCAND = S * GSZ

BQ = 256
BQ3 = 16


def _k1_body(k_ref, q_ref, gm_ref):
    kt = k_ref[...]
    q = q_ref[...]
    kn = jnp.sqrt(jnp.sum(kt * kt, axis=1, keepdims=True))
    rinv = 1.0 / jnp.maximum(kn, jnp.float32(1e-30))
    dot = jnp.dot(kt, q, preferred_element_type=jnp.float32)
    s = dot * rinv
    t = pl.program_id(0)
    row = t * TILE + lax.broadcasted_iota(jnp.int32, (TILE, 1), 0)
    s = jnp.where(row < K, s, jnp.float32(-3.0e38))
    gm_ref[...] = jnp.max(s.reshape(GPT, GSZ, BQ), axis=1)


def _k2_body(gm_ref, sel_ref):
    x = gm_ref[...]
    riota = lax.broadcasted_iota(jnp.int32, (NG, BQ), 0)
    siota = lax.broadcasted_iota(jnp.int32, (S, BQ), 0)
    sel = jnp.zeros((S, BQ), jnp.int32)
    for s in range(S):
        m = jnp.max(x, axis=0, keepdims=True)
        idx = jnp.min(jnp.where(x == m, riota, NG), axis=0, keepdims=True)
        sel = jnp.where(siota == s, idx, sel)
        x = jnp.where(riota == idx, jnp.float32(-3.0e38), x)
    sel_ref[...] = sel


def _k3b_body(q_ref, cand_ref, clab_ref, topd_ref, votes_ref, pos_ref):
    q = q_ref[...]
    ckt = cand_ref[...]
    qn = jnp.sqrt(jnp.sum(q * q, axis=1, keepdims=True))
    kn = jnp.sqrt(jnp.sum(ckt * ckt, axis=0, keepdims=True))
    dot = jnp.dot(q, ckt, preferred_element_type=jnp.float32)
    denom = jnp.maximum(qn * kn, jnp.float32(EPS))
    dist = 1.0 - dot / denom
    lane = lax.broadcasted_iota(jnp.int32, (BQ3, BQ3 * NCAND), 1)
    rowi = lax.broadcasted_iota(jnp.int32, (BQ3, BQ3 * NCAND), 0)
    own = (lane // NCAND) == rowi
    dz = jnp.where(own, dist, jnp.float32(0.0))
    x = dz[:, 0:NCAND]
    for b in range(1, BQ3):
        x = x + dz[:, b * NCAND:(b + 1) * NCAND]

    soff = lax.broadcasted_iota(jnp.int32, (BQ3, NCAND), 1)
    inf = jnp.float32(jnp.inf)
    bigi = jnp.int32(2 ** 30)
    lane16 = lax.broadcasted_iota(jnp.int32, (BQ3, 16), 1)
    topd = jnp.zeros((BQ3, 16), jnp.float32)
    hsum = jnp.zeros((BQ3, NCAND), jnp.float32)
    for t in range(TOPN):
        m = jnp.min(x, axis=1, keepdims=True)
        cs = jnp.min(jnp.where(x == m, soff, bigi), axis=1, keepdims=True)
        topd = jnp.where(lane16 == t, m, topd)
        hit = (x == m) & (soff == cs)
        hsum = hsum + jnp.where(hit, jnp.float32(1.0), jnp.float32(0.0))
        x = jnp.where(hit, inf, x)
    topd_ref[...] = topd

    hfull = jnp.concatenate([hsum] * BQ3, axis=1)
    hown = jnp.where(own, hfull, jnp.float32(0.0))
    labf = clab_ref[...].astype(jnp.float32)
    vsum = jnp.dot(hown, labf, preferred_element_type=jnp.float32)
    votes = vsum / jnp.float32(TOPN)
    votes_ref[...] = votes
    pos_ref[...] = jnp.where(votes > 0.5, jnp.float32(1.0), jnp.float32(0.0))


def _sc_params():
    return pltpu.CompilerParams(needs_layout_passes=False)


def _make_k3a(nc, ns):
    nw = nc * ns
    qpw = Q // nw
    mesh = plsc.VectorSubcoreMesh(core_axis_name="c", subcore_axis_name="s")

    @functools.partial(
        pl.kernel,
        out_type=(
            jax.ShapeDtypeStruct((Q, 64, 128), jnp.float32),
            jax.ShapeDtypeStruct((Q, 64, 128), jnp.int32),
        ),
        mesh=mesh,
        compiler_params=_sc_params(),
        scratch_types=[
            pltpu.VMEM((qpw, 16), jnp.int32),
            pltpu.VMEM((64,), jnp.int32),
            pltpu.VMEM((64, 128), jnp.float32),
            pltpu.VMEM((64, 128), jnp.int32),
            pltpu.SemaphoreType.DMA,
            pltpu.SemaphoreType.DMA,
        ],
    )
    def k3a(keys_hbm, lab_hbm, sel_hbm, cand_hbm, clab_hbm,
            sel32, cidx, cand, clab, sem, sem2):
        w = lax.axis_index("s") * nc + lax.axis_index("c")
        base_q = w * qpw
        pltpu.sync_copy(sel_hbm.at[pl.ds(base_q, qpw)], sel32)
        iota = lax.iota(jnp.int32, 16)

        def full_i(v):
            return jnp.full((16,), v, jnp.int32)

        def per_query(qi, _):
            selv = plsc.load_gather(sel32, [full_i(qi), iota])
            selv, _v = plsc.sort_key_val(selv, selv)
            for r in range(4):
                plsc.store_scatter(cidx, [iota * 4 + r], selv * 4 + r)
            cp1 = pltpu.async_copy(keys_hbm.at[cidx], cand, sem)
            cp2 = pltpu.async_copy(lab_hbm.at[cidx], clab, sem2)
            cp1.wait()
            cp2.wait()
            pltpu.sync_copy(cand, cand_hbm.at[base_q + qi])
            pltpu.sync_copy(clab, clab_hbm.at[base_q + qi])
            return 0

        lax.fori_loop(0, qpw, per_query, 0)

    return k3a


def kernel(queries, keys, train_labels):
    keys_pad = jnp.pad(keys, ((0, KPAD - K), (0, 0)))
    keys128 = keys_pad.reshape(KPAD * D // 128, 128)
    queries_t = queries.T
    lab_pad = jnp.pad(train_labels, ((0, 0), (0, 16 - C)))
    lab128 = lab_pad.reshape(K * 16 // 128, 128)

    gm = pl.pallas_call(
        _k1_body,
        grid=(NT, Q // BQ),
        in_specs=[
            pl.BlockSpec((TILE, D), lambda t, qb: (t, 0)),
            pl.BlockSpec((D, BQ), lambda t, qb: (0, qb)),
        ],
        out_specs=pl.BlockSpec((GPT, BQ), lambda t, qb: (t, qb)),
        out_shape=jax.ShapeDtypeStruct((NG, Q), jnp.float32),
    )(keys_pad, queries_t)

    sel_t = pl.pallas_call(
        _k2_body,
        grid=(Q // BQ,),
        in_specs=[pl.BlockSpec((NG, BQ), lambda qb: (0, qb))],
        out_specs=pl.BlockSpec((S, BQ), lambda qb: (0, qb)),
        out_shape=jax.ShapeDtypeStruct((S, Q), jnp.int32),
    )(gm)
    sel = sel_t.T

    try:
        info = plsc.get_sparse_core_info()
        nc, ns = info.num_cores, info.num_subcores
    except Exception:
        nc, ns = 2, 16

    cand, clab = _make_k3a(nc, ns)(keys128, lab128, sel)
    cand_t = cand.reshape(Q * NCAND, D).T
    clab_rows = clab.reshape(Q * NCAND, 16)

    topd16, votes16, pos16 = pl.pallas_call(
        _k3b_body,
        grid=(Q // BQ3,),
        in_specs=[
            pl.BlockSpec((BQ3, D), lambda b: (b, 0)),
            pl.BlockSpec((D, BQ3 * NCAND), lambda b: (0, b)),
            pl.BlockSpec((BQ3 * NCAND, 16), lambda b: (b, 0)),
        ],
        out_specs=(
            pl.BlockSpec((BQ3, 16), lambda b: (b, 0)),
            pl.BlockSpec((BQ3, 16), lambda b: (b, 0)),
            pl.BlockSpec((BQ3, 16), lambda b: (b, 0)),
        ),
        out_shape=(
            jax.ShapeDtypeStruct((Q, 16), jnp.float32),
            jax.ShapeDtypeStruct((Q, 16), jnp.float32),
            jax.ShapeDtypeStruct((Q, 16), jnp.float32),
        ),
    )(queries, cand_t, clab_rows)

    topd = topd16[:, :TOPN]
    votes = votes16[:, :C]
    pos = pos16[:, :C]
    return topd, votes, pos


# speedup vs baseline: 2.9117x; 1.0146x over previous
import functools

import jax
import jax.numpy as jnp
from jax import lax
from jax.experimental import pallas as pl
from jax.experimental.pallas import tpu as pltpu
from jax.experimental.pallas import tpu_sc as plsc

Q = 1024
K = 100000
D = 16
C = 14
TOPN = 10
EPS = 1e-8

TILE = 4096
NT = 25
KPAD = TILE * NT
GSZ = 32
GPT = TILE // GSZ
NG = NT * GPT
S = 16
NCAND = S * GSZ

BQ = 256
BQ3 = 16


def _k1_body(k_ref, q_ref, gm_ref):
    kt = k_ref[...]
    q = q_ref[...]
    kn = jnp.sqrt(jnp.sum(kt * kt, axis=1, keepdims=True))
    rinv = 1.0 / jnp.maximum(kn, jnp.float32(1e-30))
    dot = jnp.dot(kt, q, preferred_element_type=jnp.float32)
    s = dot * rinv
    t = pl.program_id(0)
    row = t * TILE + lax.broadcasted_iota(jnp.int32, (TILE, 1), 0)
    s = jnp.where(row < K, s, jnp.float32(-3.0e38))
    gm_ref[...] = jnp.max(s.reshape(GPT, GSZ, BQ), axis=1)


def _k2_body(gm_ref, sel_ref):
    x = gm_ref[...]
    riota = lax.broadcasted_iota(jnp.int32, (NG, BQ), 0)
    siota = lax.broadcasted_iota(jnp.int32, (S, BQ), 0)
    sel = jnp.zeros((S, BQ), jnp.int32)
    for s in range(S):
        m = jnp.max(x, axis=0, keepdims=True)
        idx = jnp.min(jnp.where(x == m, riota, NG), axis=0, keepdims=True)
        sel = jnp.where(siota == s, idx, sel)
        x = jnp.where(riota == idx, jnp.float32(-3.0e38), x)
    sel_ref[...] = sel


def _k3b_body(q_ref, cand_ref, clab_ref, topd_ref, votes_ref, pos_ref):
    q = q_ref[...]
    ckt = cand_ref[...]
    qn = jnp.sqrt(jnp.sum(q * q, axis=1, keepdims=True))
    kn = jnp.sqrt(jnp.sum(ckt * ckt, axis=0, keepdims=True))
    dot = jnp.dot(q, ckt, preferred_element_type=jnp.float32)
    denom = jnp.maximum(qn * kn, jnp.float32(EPS))
    dist = 1.0 - dot / denom
    lane = lax.broadcasted_iota(jnp.int32, (BQ3, BQ3 * NCAND), 1)
    rowi = lax.broadcasted_iota(jnp.int32, (BQ3, BQ3 * NCAND), 0)
    own = (lane // NCAND) == rowi
    dz = jnp.where(own, dist, jnp.float32(0.0))
    x = dz[:, 0:NCAND]
    for b in range(1, BQ3):
        x = x + dz[:, b * NCAND:(b + 1) * NCAND]

    soff = lax.broadcasted_iota(jnp.int32, (BQ3, NCAND), 1)
    inf = jnp.float32(jnp.inf)
    bigi = jnp.int32(2 ** 30)
    lane16 = lax.broadcasted_iota(jnp.int32, (BQ3, 16), 1)
    topd = jnp.zeros((BQ3, 16), jnp.float32)
    hsum = jnp.zeros((BQ3, NCAND), jnp.float32)
    for t in range(TOPN):
        m = jnp.min(x, axis=1, keepdims=True)
        cs = jnp.min(jnp.where(x == m, soff, bigi), axis=1, keepdims=True)
        topd = jnp.where(lane16 == t, m, topd)
        hit = (x == m) & (soff == cs)
        hsum = hsum + jnp.where(hit, jnp.float32(1.0), jnp.float32(0.0))
        x = jnp.where(hit, inf, x)
    topd_ref[...] = topd

    hfull = jnp.concatenate([hsum] * BQ3, axis=1)
    hown = jnp.where(own, hfull, jnp.float32(0.0))
    labf = clab_ref[...].astype(jnp.float32)
    vsum = jnp.dot(hown, labf, preferred_element_type=jnp.float32)
    votes = vsum / jnp.float32(TOPN)
    votes_ref[...] = votes
    pos_ref[...] = jnp.where(votes > 0.5, jnp.float32(1.0), jnp.float32(0.0))


def _sc_params():
    return pltpu.CompilerParams(needs_layout_passes=False)


def _make_k3a(nc, ns):
    nw = nc * ns
    qpw = Q // nw
    mesh = plsc.VectorSubcoreMesh(core_axis_name="c", subcore_axis_name="s")

    @functools.partial(
        pl.kernel,
        out_type=(
            jax.ShapeDtypeStruct((Q, 64, 128), jnp.float32),
            jax.ShapeDtypeStruct((Q, 64, 128), jnp.int32),
        ),
        mesh=mesh,
        compiler_params=_sc_params(),
        scratch_types=[
            pltpu.VMEM((qpw, 16), jnp.int32),
            pltpu.VMEM((64,), jnp.int32),
            pltpu.VMEM((64,), jnp.int32),
            pltpu.VMEM((64, 128), jnp.float32),
            pltpu.VMEM((64, 128), jnp.float32),
            pltpu.VMEM((64, 128), jnp.int32),
            pltpu.VMEM((64, 128), jnp.int32),
            pltpu.SemaphoreType.DMA,
            pltpu.SemaphoreType.DMA,
            pltpu.SemaphoreType.DMA,
            pltpu.SemaphoreType.DMA,
            pltpu.SemaphoreType.DMA,
            pltpu.SemaphoreType.DMA,
            pltpu.SemaphoreType.DMA,
            pltpu.SemaphoreType.DMA,
        ],
    )
    def k3a(keys_hbm, lab_hbm, sel_hbm, cand_hbm, clab_hbm,
            sel32, cidx0, cidx1, cand0, cand1, clab0, clab1,
            gk0, gl0, gk1, gl1, ok0, ol0, ok1, ol1):
        w = lax.axis_index("s") * nc + lax.axis_index("c")
        base_q = w * qpw
        pltpu.sync_copy(sel_hbm.at[pl.ds(base_q, qpw)], sel32)
        iota = lax.iota(jnp.int32, 16)

        def full_i(v):
            return jnp.full((16,), v, jnp.int32)

        cidx = [cidx0, cidx1]
        cand = [cand0, cand1]
        clab = [clab0, clab1]
        gsem = [(gk0, gl0), (gk1, gl1)]
        osem = [(ok0, ol0), (ok1, ol1)]

        def fire_gather(qi, b):
            selv = plsc.load_gather(sel32, [full_i(qi), iota])
            selv, _v = plsc.sort_key_val(selv, selv)
            for r in range(4):
                plsc.store_scatter(cidx[b], [iota * 4 + r], selv * 4 + r)
            return (pltpu.async_copy(keys_hbm.at[cidx[b]], cand[b], gsem[b][0]),
                    pltpu.async_copy(lab_hbm.at[cidx[b]], clab[b], gsem[b][1]))

        gcp = [None, None]
        ocp = [None, None]
        gcp[0] = fire_gather(0, 0)
        for qi in range(qpw):
            b = qi % 2
            o = 1 - b
            gcp[b][0].wait()
            gcp[b][1].wait()
            ocp[b] = (pltpu.async_copy(cand[b], cand_hbm.at[base_q + qi],
                                       osem[b][0]),
                      pltpu.async_copy(clab[b], clab_hbm.at[base_q + qi],
                                       osem[b][1]))
            if qi + 1 < qpw:
                if ocp[o] is not None:
                    ocp[o][0].wait()
                    ocp[o][1].wait()
                    ocp[o] = None
                gcp[o] = fire_gather(qi + 1, o)
        for b in range(2):
            if ocp[b] is not None:
                ocp[b][0].wait()
                ocp[b][1].wait()

    return k3a


def kernel(queries, keys, train_labels):
    keys_pad = jnp.pad(keys, ((0, KPAD - K), (0, 0)))
    keys128 = keys_pad.reshape(KPAD * D // 128, 128)
    queries_t = queries.T
    lab_pad = jnp.pad(train_labels, ((0, 0), (0, 16 - C)))
    lab128 = lab_pad.reshape(K * 16 // 128, 128)

    gm = pl.pallas_call(
        _k1_body,
        grid=(NT, Q // BQ),
        in_specs=[
            pl.BlockSpec((TILE, D), lambda t, qb: (t, 0)),
            pl.BlockSpec((D, BQ), lambda t, qb: (0, qb)),
        ],
        out_specs=pl.BlockSpec((GPT, BQ), lambda t, qb: (t, qb)),
        out_shape=jax.ShapeDtypeStruct((NG, Q), jnp.float32),
    )(keys_pad, queries_t)

    sel_t = pl.pallas_call(
        _k2_body,
        grid=(Q // BQ,),
        in_specs=[pl.BlockSpec((NG, BQ), lambda qb: (0, qb))],
        out_specs=pl.BlockSpec((S, BQ), lambda qb: (0, qb)),
        out_shape=jax.ShapeDtypeStruct((S, Q), jnp.int32),
    )(gm)
    sel = sel_t.T

    try:
        info = plsc.get_sparse_core_info()
        nc, ns = info.num_cores, info.num_subcores
    except Exception:
        nc, ns = 2, 16

    cand, clab = _make_k3a(nc, ns)(keys128, lab128, sel)
    cand_t = cand.reshape(Q * NCAND, D).T
    clab_rows = clab.reshape(Q * NCAND, 16)

    topd16, votes16, pos16 = pl.pallas_call(
        _k3b_body,
        grid=(Q // BQ3,),
        in_specs=[
            pl.BlockSpec((BQ3, D), lambda b: (b, 0)),
            pl.BlockSpec((D, BQ3 * NCAND), lambda b: (0, b)),
            pl.BlockSpec((BQ3 * NCAND, 16), lambda b: (b, 0)),
        ],
        out_specs=(
            pl.BlockSpec((BQ3, 16), lambda b: (b, 0)),
            pl.BlockSpec((BQ3, 16), lambda b: (b, 0)),
            pl.BlockSpec((BQ3, 16), lambda b: (b, 0)),
        ),
        out_shape=(
            jax.ShapeDtypeStruct((Q, 16), jnp.float32),
            jax.ShapeDtypeStruct((Q, 16), jnp.float32),
            jax.ShapeDtypeStruct((Q, 16), jnp.float32),
        ),
    )(queries, cand_t, clab_rows)

    topd = topd16[:, :TOPN]
    votes = votes16[:, :C]
    pos = pos16[:, :C]
    return topd, votes, pos
